# Initial kernel scaffold; baseline (speedup 1.0000x reference)
import jax
import jax.numpy as jnp
from jax.experimental import pallas as pl


def kernel(bias_table, query_len, cond_len):
    raise NotImplementedError("write your pallas kernel here")


# speedup vs baseline: 1334.8257x; 1334.8257x over previous
import functools

import jax
import jax.numpy as jnp
from jax import lax
from jax.experimental import pallas as pl
from jax.experimental.pallas import tpu as pltpu
from jax.experimental.pallas import tpu_sc as plsc

_MAXD = 2048
_N = 4096
_T = 2 * _MAXD - 1
_EREV = 8192
_NSHIFT = 16
_NW = 32
_ROWS = _N // _NW
_CHUNK = _ROWS - _NSHIFT + _N


def _body(erevx_hbm, out_hbm, chunkx, ldsem, stsem):
    wid = lax.axis_index("c") * 16 + lax.axis_index("s")
    i0 = wid * _ROWS
    start_min = (_N - _ROWS) - i0

    loads = [
        pltpu.async_copy(
            erevx_hbm.at[pl.ds(_EREV * c + start_min, _CHUNK)],
            chunkx.at[pl.ds(_CHUNK * c, _CHUNK)],
            ldsem,
        )
        for c in range(_NSHIFT)
    ]
    for h in loads:
        h.wait()

    batches = []
    for c in range(_NSHIFT):
        batch = [
            pltpu.async_copy(
                chunkx.at[pl.ds(_CHUNK * c + 16 * t, _N)],
                out_hbm.at[pl.ds((i0 + _ROWS - 1 - c - 16 * t) * _N, _N)],
                stsem,
            )
            for t in range(_ROWS // _NSHIFT)
        ]
        batches.append(batch)
        if c >= 1:
            for h in batches[c - 1]:
                h.wait()
    for h in batches[-1]:
        h.wait()


def _toeplitz_rows(erevx):
    mesh = plsc.VectorSubcoreMesh(core_axis_name="c", subcore_axis_name="s")
    f = functools.partial(
        pl.kernel,
        mesh=mesh,
        out_type=jax.ShapeDtypeStruct((_N * _N,), jnp.float32),
        scratch_types=[
            pltpu.VMEM((_NSHIFT * _CHUNK,), jnp.float32),
            pltpu.SemaphoreType.DMA,
            pltpu.SemaphoreType.DMA,
        ],
    )(_body)
    return f(erevx).reshape(_N, _N)


def kernel(bias_table, query_len, cond_len):
    d = jnp.asarray(query_len, jnp.int32) - jnp.asarray(cond_len, jnp.int32)
    base = jnp.pad(jnp.flip(bias_table), (_N, _N), mode="edge")
    dc = jnp.clip(d, -2000, 2000)
    big = lax.dynamic_slice(base, (_MAXD - dc,), (_EREV + _NSHIFT,))
    erevx = jnp.concatenate([big[c : c + _EREV] for c in range(_NSHIFT)])
    return _toeplitz_rows(erevx)
